# Initial kernel scaffold; baseline (speedup 1.0000x reference)
import functools

import jax
import jax.numpy as jnp
from jax import lax
from jax.experimental import pallas as pl
from jax.experimental.pallas import tpu as pltpu
from jax.experimental.pallas import tpu_sc as plsc

B, S, T = 16, 512, 128
NC, NS, L = 2, 16, 16
NW = NC * NS
NCHUNK = S // L
PAD = 8

_mesh = plsc.VectorSubcoreMesh(core_axis_name="c", subcore_axis_name="s")


@functools.partial(
    pl.kernel,
    out_type=jax.ShapeDtypeStruct((NW, L), jnp.float32),
    mesh=_mesh,
    scratch_types=[
        pltpu.VMEM((PAD + S,), jnp.int32),
        pltpu.VMEM((4, 128), jnp.int32),
        pltpu.VMEM((4, 128), jnp.int32),
        pltpu.VMEM((4, 128), jnp.float32),
        pltpu.VMEM((4, 128), jnp.float32),
        pltpu.VMEM((L,), jnp.float32),
        pltpu.SemaphoreType.DMA,
        pltpu.SemaphoreType.DMA,
    ],
)
def _score_kernel(tgs_hbm, unary_hbm, binary_hbm, out_hbm,
                  tag_v, idxu_v, idxb_v, uval_v, bval_v, sval_v,
                  sem_u, sem_b):
    wid = lax.axis_index("s") * NC + lax.axis_index("c")
    b = wid % B
    lanes = lax.iota(jnp.int32, L)

    tag_v[pl.ds(0, L)] = jnp.zeros((L,), jnp.int32)
    pltpu.sync_copy(tgs_hbm.at[wid], tag_v.at[pl.ds(PAD, S)])

    base_u = b * (S * T)
    for j in range(NCHUNK):
        t = tag_v[pl.ds(PAD + j * L, L)]
        tp = tag_v[pl.ds(PAD - 1 + j * L, L)]
        iu = base_u + (j * L + lanes) * T + t
        ib = tp * T + t
        if j == 0:
            ib = jnp.where(lanes == 0, 0, ib)
        r, c = j // 8, (j % 8) * L
        idxu_v[r, pl.ds(c, L)] = iu
        idxb_v[r, pl.ds(c, L)] = ib

    cu = pltpu.async_copy(unary_hbm.at[idxu_v], uval_v, sem_u)
    cb = pltpu.async_copy(binary_hbm.at[idxb_v], bval_v, sem_b)
    cu.wait()
    cb.wait()

    acc = jnp.zeros((L,), jnp.float32)
    for j in range(NCHUNK):
        r, c = j // 8, (j % 8) * L
        u = uval_v[r, pl.ds(c, L)]
        v = bval_v[r, pl.ds(c, L)]
        if j == 0:
            v = jnp.where(lanes == 0, jnp.float32(0.0), v)
        acc = acc + u + v
    score = jnp.sum(acc)
    sval_v[...] = jnp.broadcast_to(score, (L,))
    pltpu.sync_copy(sval_v, out_hbm.at[wid])


@functools.partial(
    pl.kernel,
    out_type=jax.ShapeDtypeStruct((L,), jnp.float32),
    mesh=_mesh,
    scratch_types=[
        pltpu.VMEM((NW, L), jnp.float32),
        pltpu.VMEM((L,), jnp.float32),
    ],
)
def _loss_kernel(scores_hbm, out_hbm, buf_v, res_v):
    wid = lax.axis_index("s") * NC + lax.axis_index("c")

    @pl.when(wid == 0)
    def _():
        pltpu.sync_copy(scores_hbm, buf_v)
        acc = jnp.zeros((L,), jnp.float32)
        for bb in range(B):
            g = buf_v[bb, pl.ds(0, L)]
            p = buf_v[B + bb, pl.ds(0, L)]
            acc = acc + jnp.maximum(p - g, jnp.float32(0.0))
        res_v[...] = acc
        pltpu.sync_copy(res_v, out_hbm)


def kernel(unary_potentials, binary_potentials, tags, predicted_tags, mask):
    tgs = jnp.concatenate(
        [tags.astype(jnp.int32), predicted_tags.astype(jnp.int32)], axis=0
    )
    scores = _score_kernel(
        tgs, unary_potentials.reshape(-1), binary_potentials.reshape(-1)
    )
    out = _loss_kernel(scores)
    return out[0]


# speedup vs baseline: 1.7636x; 1.7636x over previous
import functools

import jax
import jax.numpy as jnp
from jax import lax
from jax.experimental import pallas as pl
from jax.experimental.pallas import tpu as pltpu
from jax.experimental.pallas import tpu_sc as plsc

B, S, T = 16, 512, 128
NC, NS, L = 2, 16, 16
NW = NC * NS
NCHUNK = S // L
PAD = 8

_mesh = plsc.VectorSubcoreMesh(core_axis_name="c", subcore_axis_name="s")


def _lane_total(v):
    s = v[0]
    for i in range(1, L):
        s = s + v[i]
    return jnp.broadcast_to(s, (L,))


@functools.partial(
    pl.kernel,
    out_type=jax.ShapeDtypeStruct((NW * L,), jnp.float32),
    mesh=_mesh,
    scratch_types=[
        pltpu.VMEM((PAD + S,), jnp.int32),
        pltpu.VMEM((4, 128), jnp.int32),
        pltpu.VMEM((4, 128), jnp.int32),
        pltpu.VMEM((4, 128), jnp.float32),
        pltpu.VMEM((4, 128), jnp.float32),
        pltpu.VMEM((L,), jnp.float32),
        pltpu.SemaphoreType.DMA,
        pltpu.SemaphoreType.DMA,
    ],
)
def _score_kernel(tgs_hbm, unary_hbm, binary_hbm, out_hbm,
                  tag_v, idxu_v, idxb_v, uval_v, bval_v, sval_v,
                  sem_u, sem_b):
    wid = lax.axis_index("s") * NC + lax.axis_index("c")
    b = wid % B
    lanes = lax.iota(jnp.int32, L)

    tag_v[pl.ds(0, L)] = jnp.zeros((L,), jnp.int32)
    pltpu.sync_copy(tgs_hbm.at[pl.ds(wid * S, S)], tag_v.at[pl.ds(PAD, S)])

    base_u = b * (S * T)
    for j in range(NCHUNK):
        t = tag_v[pl.ds(PAD + j * L, L)]
        tp = tag_v[pl.ds(PAD - 1 + j * L, L)]
        iu = base_u + (j * L + lanes) * T + t
        ib = tp * T + t
        if j == 0:
            ib = jnp.where(lanes == 0, 0, ib)
        r, c = j // 8, (j % 8) * L
        idxu_v[r, pl.ds(c, L)] = iu
        idxb_v[r, pl.ds(c, L)] = ib

    copies = []
    for r in range(4):
        copies.append(
            pltpu.async_copy(unary_hbm.at[idxu_v.at[r]], uval_v.at[r], sem_u))
        copies.append(
            pltpu.async_copy(binary_hbm.at[idxb_v.at[r]], bval_v.at[r], sem_b))
    for cp in copies:
        cp.wait()

    acc = jnp.zeros((L,), jnp.float32)
    for j in range(NCHUNK):
        r, c = j // 8, (j % 8) * L
        u = uval_v[r, pl.ds(c, L)]
        v = bval_v[r, pl.ds(c, L)]
        if j == 0:
            v = jnp.where(lanes == 0, jnp.float32(0.0), v)
        acc = acc + u + v
    sval_v[...] = _lane_total(acc)
    pltpu.sync_copy(sval_v, out_hbm.at[pl.ds(wid * L, L)])


@functools.partial(
    pl.kernel,
    out_type=jax.ShapeDtypeStruct((L,), jnp.float32),
    mesh=_mesh,
    scratch_types=[
        pltpu.VMEM((NW * L,), jnp.float32),
        pltpu.VMEM((L,), jnp.float32),
    ],
)
def _loss_kernel(scores_hbm, out_hbm, buf_v, res_v):
    wid = lax.axis_index("s") * NC + lax.axis_index("c")

    @pl.when(wid == 0)
    def _():
        pltpu.sync_copy(scores_hbm, buf_v)
        acc = jnp.zeros((L,), jnp.float32)
        for bb in range(B):
            g = buf_v[pl.ds(bb * L, L)]
            p = buf_v[pl.ds((B + bb) * L, L)]
            acc = acc + jnp.maximum(p - g, jnp.float32(0.0))
        res_v[...] = acc
        pltpu.sync_copy(res_v, out_hbm)


def kernel(unary_potentials, binary_potentials, tags, predicted_tags, mask):
    tgs = jnp.concatenate(
        [tags.astype(jnp.int32), predicted_tags.astype(jnp.int32)], axis=0
    ).reshape(-1)
    scores = _score_kernel(
        tgs, unary_potentials.reshape(-1), binary_potentials.reshape(-1)
    )
    out = _loss_kernel(scores)
    return out[0]


# speedup vs baseline: 1.8280x; 1.0365x over previous
import functools

import jax
import jax.numpy as jnp
from jax import lax
from jax.experimental import pallas as pl
from jax.experimental.pallas import tpu as pltpu
from jax.experimental.pallas import tpu_sc as plsc

B, S, T = 16, 512, 128
NC, NS, L = 2, 16, 16
NW = NC * NS
NCHUNK = S // L
PAD = 8

_mesh = plsc.VectorSubcoreMesh(core_axis_name="c", subcore_axis_name="s")


def _lane_total(v):
    s = v[0]
    for i in range(1, L):
        s = s + v[i]
    return jnp.broadcast_to(s, (L,))


@functools.partial(
    pl.kernel,
    out_type=jax.ShapeDtypeStruct((NC * L,), jnp.float32),
    mesh=_mesh,
    scratch_types=[
        pltpu.VMEM((PAD + S,), jnp.int32),
        pltpu.VMEM((4, 128), jnp.int32),
        pltpu.VMEM((4, 128), jnp.int32),
        pltpu.VMEM((4, 128), jnp.float32),
        pltpu.VMEM((4, 128), jnp.float32),
        pltpu.VMEM((L,), jnp.float32),
        pltpu.VMEM((NS * L,), jnp.float32),
        pltpu.VMEM_SHARED((NS * L,), jnp.float32),
        pltpu.SemaphoreType.DMA,
        pltpu.SemaphoreType.DMA,
    ],
)
def _loss_sc_kernel(tgs_hbm, unary_hbm, binary_hbm, out_hbm,
                    tag_v, idxu_v, idxb_v, uval_v, bval_v, sval_v, buf_v,
                    shared_v, sem_u, sem_b):
    c = lax.axis_index("c")
    s = lax.axis_index("s")
    which = s % 2
    b = c * (B // NC) + s // 2
    row = which * B + b
    lanes = lax.iota(jnp.int32, L)

    tag_v[pl.ds(0, L)] = jnp.zeros((L,), jnp.int32)
    pltpu.sync_copy(tgs_hbm.at[pl.ds(row * S, S)], tag_v.at[pl.ds(PAD, S)])

    base_u = b * (S * T)
    for j in range(NCHUNK):
        t = tag_v[pl.ds(PAD + j * L, L)]
        tp = tag_v[pl.ds(PAD - 1 + j * L, L)]
        iu = base_u + (j * L + lanes) * T + t
        ib = tp * T + t
        if j == 0:
            ib = jnp.where(lanes == 0, 0, ib)
        r, col = j // 8, (j % 8) * L
        idxu_v[r, pl.ds(col, L)] = iu
        idxb_v[r, pl.ds(col, L)] = ib

    copies = []
    for r in range(4):
        copies.append(
            pltpu.async_copy(unary_hbm.at[idxu_v.at[r]], uval_v.at[r], sem_u))
        copies.append(
            pltpu.async_copy(binary_hbm.at[idxb_v.at[r]], bval_v.at[r], sem_b))
    for cp in copies:
        cp.wait()

    acc = jnp.zeros((L,), jnp.float32)
    for j in range(NCHUNK):
        r, col = j // 8, (j % 8) * L
        u = uval_v[r, pl.ds(col, L)]
        v = bval_v[r, pl.ds(col, L)]
        if j == 0:
            v = jnp.where(lanes == 0, jnp.float32(0.0), v)
        acc = acc + u + v
    sval_v[...] = _lane_total(acc)
    pltpu.sync_copy(sval_v, shared_v.at[pl.ds(s * L, L)])
    plsc.subcore_barrier()

    @pl.when(s == 0)
    def _():
        pltpu.sync_copy(shared_v, buf_v)
        red = jnp.zeros((L,), jnp.float32)
        for k in range(NS // 2):
            g = buf_v[pl.ds((2 * k) * L, L)]
            p = buf_v[pl.ds((2 * k + 1) * L, L)]
            red = red + jnp.maximum(p - g, jnp.float32(0.0))
        sval_v[...] = red
        pltpu.sync_copy(sval_v, out_hbm.at[pl.ds(c * L, L)])


def kernel(unary_potentials, binary_potentials, tags, predicted_tags, mask):
    tgs = jnp.concatenate(
        [tags.astype(jnp.int32), predicted_tags.astype(jnp.int32)], axis=0
    ).reshape(-1)
    out = _loss_sc_kernel(
        tgs, unary_potentials.reshape(-1), binary_potentials.reshape(-1)
    )
    return out[0] + out[L]
